# baseline (speedup 1.0000x reference)
import functools
import jax
import jax.numpy as jnp
from jax import lax
from jax.experimental import pallas as pl
from jax.experimental.pallas import tpu as pltpu
from jax.experimental.pallas import tpu_sc as plsc

_N = 1000000
_V = 32
_L = 16
_NVEC = _N // _L
_NC = 2
_NS = 16
_NW = _NC * _NS
_TV = 64
_C = _TV * _L

_mesh = plsc.VectorSubcoreMesh(core_axis_name="c", subcore_axis_name="s")


@functools.partial(
    pl.kernel,
    mesh=_mesh,
    out_type=jax.ShapeDtypeStruct((_N,), jnp.int32),
    scratch_types=[
        pltpu.VMEM((_V, _C), jnp.int32),
        pltpu.VMEM((_C,), jnp.int32),
    ],
    compiler_params=pltpu.CompilerParams(use_tc_tiling_on_sc=False),
)
def _sc_vote(in_hbm, out_hbm, in_v, out_v):
    wid = lax.axis_index("s") * _NC + lax.axis_index("c")
    vbase = (wid * _NVEC) // _NW
    vend = ((wid + 1) * _NVEC) // _NW
    cnt = vend - vbase
    ntiles = (cnt + _TV - 1) // _TV

    def tile_body(t, carry):
        vb = jnp.minimum(vbase + t * _TV, vend - _TV)
        col = vb * _L
        pltpu.sync_copy(in_hbm.at[:, pl.ds(col, _C)], in_v)

        def vec_body(j, carry2):
            acc = in_v[0, pl.ds(j * _L, _L)]
            for v in range(1, _V):
                acc = acc + in_v[v, pl.ds(j * _L, _L)]
            out_v[pl.ds(j * _L, _L)] = jnp.where(
                acc > _V // 2, jnp.int32(1), jnp.int32(0)
            )
            return carry2

        lax.fori_loop(0, _TV, vec_body, 0, unroll=4)
        pltpu.sync_copy(out_v, out_hbm.at[pl.ds(col, _C)])
        return carry

    lax.fori_loop(0, ntiles, tile_body, 0)


def kernel(inputs):
    return _sc_vote(inputs)


# speedup vs baseline: 1.0168x; 1.0168x over previous
import functools
import jax
import jax.numpy as jnp
from jax import lax
from jax.experimental import pallas as pl
from jax.experimental.pallas import tpu as pltpu
from jax.experimental.pallas import tpu_sc as plsc

_N = 1000000
_V = 32
_L = 16
_NVEC = _N // _L
_NC = 2
_NS = 16
_NW = _NC * _NS
_TV = 96
_C = _TV * _L

_mesh = plsc.VectorSubcoreMesh(core_axis_name="c", subcore_axis_name="s")


@functools.partial(
    pl.kernel,
    mesh=_mesh,
    out_type=jax.ShapeDtypeStruct((_N,), jnp.int32),
    scratch_types=[
        pltpu.VMEM((2, _V, _C), jnp.int32),
        pltpu.VMEM((2, _C), jnp.int32),
        pltpu.SemaphoreType.DMA((2,)),
        pltpu.SemaphoreType.DMA((2,)),
    ],
    compiler_params=pltpu.CompilerParams(use_tc_tiling_on_sc=False),
)
def _sc_vote(in_hbm, out_hbm, in_v, out_v, in_sem, out_sem):
    wid = lax.axis_index("s") * _NC + lax.axis_index("c")
    vbase = (wid * _NVEC) // _NW
    vend = ((wid + 1) * _NVEC) // _NW
    cnt = vend - vbase
    ntiles = (cnt + _TV - 1) // _TV

    def col_of(t):
        return jnp.minimum(vbase + t * _TV, vend - _TV) * _L

    def start_in(t, slot):
        col = col_of(t)
        for v in range(_V):
            pltpu.make_async_copy(
                in_hbm.at[v, pl.ds(col, _C)], in_v.at[slot, v], in_sem.at[slot]
            ).start()

    def wait_in(t, slot):
        col = col_of(t)
        for v in range(_V):
            pltpu.make_async_copy(
                in_hbm.at[v, pl.ds(col, _C)], in_v.at[slot, v], in_sem.at[slot]
            ).wait()

    start_in(0, 0)

    def tile_body(t, carry):
        slot = t % 2

        @pl.when(t + 1 < ntiles)
        def _():
            start_in(t + 1, 1 - slot)

        @pl.when(t >= 2)
        def _():
            pltpu.make_async_copy(
                out_v.at[slot], out_hbm.at[pl.ds(col_of(t - 2), _C)],
                out_sem.at[slot],
            ).wait()

        wait_in(t, slot)

        def vec_body(j, carry2):
            acc = in_v[slot, 0, pl.ds(j * _L, _L)]
            for v in range(1, _V):
                acc = acc + in_v[slot, v, pl.ds(j * _L, _L)]
            out_v[slot, pl.ds(j * _L, _L)] = jnp.where(
                acc > _V // 2, jnp.int32(1), jnp.int32(0)
            )
            return carry2

        lax.fori_loop(0, _TV, vec_body, 0, unroll=4)
        pltpu.make_async_copy(
            out_v.at[slot], out_hbm.at[pl.ds(col_of(t), _C)], out_sem.at[slot]
        ).start()
        return carry

    lax.fori_loop(0, ntiles, tile_body, 0)

    @pl.when(ntiles >= 2)
    def _():
        slot = (ntiles - 2) % 2
        pltpu.make_async_copy(
            out_v.at[slot], out_hbm.at[pl.ds(col_of(ntiles - 2), _C)],
            out_sem.at[slot],
        ).wait()

    slot = (ntiles - 1) % 2
    pltpu.make_async_copy(
        out_v.at[slot], out_hbm.at[pl.ds(col_of(ntiles - 1), _C)],
        out_sem.at[slot],
    ).wait()


def kernel(inputs):
    return _sc_vote(inputs)


# speedup vs baseline: 1.0188x; 1.0020x over previous
import functools
import jax
import jax.numpy as jnp
from jax import lax
from jax.experimental import pallas as pl
from jax.experimental.pallas import tpu as pltpu
from jax.experimental.pallas import tpu_sc as plsc

_N = 1000000
_V = 32
_L = 16
_NVEC = _N // _L
_NC = 2
_NS = 16
_NW = _NC * _NS
_TV = 96
_C = _TV * _L

_mesh = plsc.VectorSubcoreMesh(core_axis_name="c", subcore_axis_name="s")


@functools.partial(
    pl.kernel,
    mesh=_mesh,
    out_type=jax.ShapeDtypeStruct((_N,), jnp.int32),
    scratch_types=[
        pltpu.VMEM((2, _V, _C), jnp.int32),
        pltpu.VMEM((2, _C), jnp.int32),
        pltpu.SemaphoreType.DMA((2,)),
        pltpu.SemaphoreType.DMA((2,)),
    ],
    compiler_params=pltpu.CompilerParams(use_tc_tiling_on_sc=False),
)
def _sc_vote(in_hbm, out_hbm, in_v, out_v, in_sem, out_sem):
    wid = lax.axis_index("s") * _NC + lax.axis_index("c")
    vbase = (wid * _NVEC) // _NW
    vend = ((wid + 1) * _NVEC) // _NW
    cnt = vend - vbase
    ntiles = (cnt + _TV - 1) // _TV

    def col_of(t):
        return jnp.minimum(vbase + t * _TV, vend - _TV) * _L

    def start_in(t, slot):
        col = col_of(t)
        for v in range(_V):
            pltpu.make_async_copy(
                in_hbm.at[v, pl.ds(col, _C)], in_v.at[slot, v], in_sem.at[slot]
            ).start()

    def wait_in(t, slot):
        col = col_of(t)
        for v in range(_V):
            pltpu.make_async_copy(
                in_hbm.at[v, pl.ds(col, _C)], in_v.at[slot, v], in_sem.at[slot]
            ).wait()

    start_in(0, 0)

    def tile_body(t, carry):
        slot = t % 2

        @pl.when(t + 1 < ntiles)
        def _():
            start_in(t + 1, 1 - slot)

        @pl.when(t >= 2)
        def _():
            pltpu.make_async_copy(
                out_v.at[slot], out_hbm.at[pl.ds(col_of(t - 2), _C)],
                out_sem.at[slot],
            ).wait()

        wait_in(t, slot)

        @plsc.parallel_loop(0, _TV, 1, unroll=4)
        def vec_body(j):
            vals = [in_v[slot, v, pl.ds(j * _L, _L)] for v in range(_V)]
            while len(vals) > 1:
                vals = [
                    vals[2 * i] + vals[2 * i + 1] for i in range(len(vals) // 2)
                ]
            out_v[slot, pl.ds(j * _L, _L)] = jnp.where(
                vals[0] > _V // 2, jnp.int32(1), jnp.int32(0)
            )
        pltpu.make_async_copy(
            out_v.at[slot], out_hbm.at[pl.ds(col_of(t), _C)], out_sem.at[slot]
        ).start()
        return carry

    lax.fori_loop(0, ntiles, tile_body, 0)

    @pl.when(ntiles >= 2)
    def _():
        slot = (ntiles - 2) % 2
        pltpu.make_async_copy(
            out_v.at[slot], out_hbm.at[pl.ds(col_of(ntiles - 2), _C)],
            out_sem.at[slot],
        ).wait()

    slot = (ntiles - 1) % 2
    pltpu.make_async_copy(
        out_v.at[slot], out_hbm.at[pl.ds(col_of(ntiles - 1), _C)],
        out_sem.at[slot],
    ).wait()


def kernel(inputs):
    return _sc_vote(inputs)
